# initial kernel scaffold (unmeasured)
import jax
import jax.numpy as jnp
from jax import lax
from jax.experimental import pallas as pl
from jax.experimental.pallas import tpu as pltpu

N_DEV = 32
BLK = 32


def kernel(x, w_mat):
    m, k_shard = x.shape
    k, n = w_mat.shape
    assert k_shard == BLK and m == N_DEV * BLK

    def body(x_ref, w_ref, out_ref, gather_ref, send_sems, recv_sems):
        my_i = lax.axis_index("i")

        sends = []
        for step in range(1, N_DEV):
            dst = lax.rem(my_i + step, N_DEV)
            rdma = pltpu.make_async_remote_copy(
                src_ref=x_ref.at[pl.ds(dst * BLK, BLK), :],
                dst_ref=gather_ref.at[my_i],
                send_sem=send_sems.at[step],
                recv_sem=recv_sems.at[step],
                device_id=(dst,),
                device_id_type=pl.DeviceIdType.MESH,
            )
            rdma.start()
            sends.append(rdma)

        gather_ref[my_i] = x_ref[pl.ds(my_i * BLK, BLK), :]

        for step in range(1, N_DEV):
            src = lax.rem(my_i + N_DEV - step, N_DEV)
            recv = pltpu.make_async_remote_copy(
                src_ref=x_ref.at[pl.ds(0, BLK), :],
                dst_ref=gather_ref.at[src],
                send_sem=send_sems.at[step],
                recv_sem=recv_sems.at[step],
                device_id=(my_i,),
                device_id_type=pl.DeviceIdType.MESH,
            )
            recv.wait_recv()
        for rdma in sends:
            rdma.wait_send()

        g = gather_ref[...]
        xrow = jnp.transpose(g, (1, 0, 2)).reshape(BLK, k)
        y = jnp.dot(xrow, w_ref[...], preferred_element_type=jnp.float32)
        out_ref[...] = y * jax.nn.sigmoid(y)

    return pl.pallas_call(
        body,
        out_shape=jax.ShapeDtypeStruct((BLK, n), jnp.float32),
        in_specs=[
            pl.BlockSpec(memory_space=pltpu.VMEM),
            pl.BlockSpec(memory_space=pltpu.VMEM),
        ],
        out_specs=pl.BlockSpec(memory_space=pltpu.VMEM),
        scratch_shapes=[
            pltpu.VMEM((N_DEV, BLK, BLK), jnp.float32),
            pltpu.SemaphoreType.DMA((N_DEV,)),
            pltpu.SemaphoreType.DMA((N_DEV,)),
        ],
        compiler_params=pltpu.CompilerParams(collective_id=0),
    )(x, w_mat)


# baseline (device time: 24140 ns/iter reference)
import jax
import jax.numpy as jnp
from jax import lax
from jax.experimental import pallas as pl
from jax.experimental.pallas import tpu as pltpu

N_DEV = 32
BLK = 32


def kernel(x, w_mat):
    m, k_shard = x.shape
    k, n = w_mat.shape
    assert k_shard == BLK and m == N_DEV * BLK

    def body(x_ref, w_ref, out_ref, gather_ref, send_sems, recv_sems):
        my_i = lax.axis_index("i")

        sends = []
        for step in range(1, N_DEV):
            dst = lax.rem(my_i + step, N_DEV)
            rdma = pltpu.make_async_remote_copy(
                src_ref=x_ref.at[pl.ds(dst * BLK, BLK), :],
                dst_ref=gather_ref.at[my_i],
                send_sem=send_sems.at[step],
                recv_sem=recv_sems.at[step],
                device_id=(dst,),
                device_id_type=pl.DeviceIdType.MESH,
            )
            rdma.start()
            sends.append(rdma)

        gather_ref[my_i] = x_ref[pl.ds(my_i * BLK, BLK), :]

        for step in range(1, N_DEV):
            src = lax.rem(my_i + N_DEV - step, N_DEV)
            recv = pltpu.make_async_remote_copy(
                src_ref=x_ref.at[pl.ds(0, BLK), :],
                dst_ref=gather_ref.at[src],
                send_sem=send_sems.at[step],
                recv_sem=recv_sems.at[step],
                device_id=(my_i,),
                device_id_type=pl.DeviceIdType.MESH,
            )
            recv.wait_recv()
        for rdma in sends:
            rdma.wait_send()

        g = gather_ref[...]
        xrow = jnp.transpose(g, (1, 0, 2)).reshape(BLK, k)
        y = jnp.dot(xrow, w_ref[...], preferred_element_type=jnp.float32)
        out_ref[...] = y * jax.nn.sigmoid(y)

    return pl.pallas_call(
        body,
        out_shape=jax.ShapeDtypeStruct((BLK, n), jnp.float32),
        in_specs=[
            pl.BlockSpec(memory_space=pltpu.VMEM),
            pl.BlockSpec(memory_space=pltpu.VMEM),
        ],
        out_specs=pl.BlockSpec(memory_space=pltpu.VMEM),
        scratch_shapes=[
            pltpu.VMEM((N_DEV, BLK, BLK), jnp.float32),
            pltpu.SemaphoreType.DMA((N_DEV,)),
            pltpu.SemaphoreType.DMA((N_DEV,)),
        ],
    )(x, w_mat)


# device time: 20540 ns/iter; 1.1753x vs baseline; 1.1753x over previous
import jax
import jax.numpy as jnp
from jax import lax
from jax.experimental import pallas as pl
from jax.experimental.pallas import tpu as pltpu

N_DEV = 32
BLK = 32


def kernel(x, w_mat):
    m, k_shard = x.shape
    k, n = w_mat.shape
    assert k_shard == BLK and m == N_DEV * BLK

    def body(
        x_ref,
        w_hbm_ref,
        out_ref,
        gather_ref,
        w_vmem_ref,
        send_sems,
        recv_sems,
        w_sem,
    ):
        my_i = lax.axis_index("i")

        w_copy = pltpu.make_async_copy(w_hbm_ref, w_vmem_ref, w_sem)
        w_copy.start()

        barrier_sem = pltpu.get_barrier_semaphore()
        for step in range(1, N_DEV):
            pl.semaphore_signal(
                barrier_sem,
                inc=1,
                device_id=(lax.rem(my_i + step, N_DEV),),
                device_id_type=pl.DeviceIdType.MESH,
            )
        pl.semaphore_wait(barrier_sem, N_DEV - 1)

        sends = []
        for step in range(1, N_DEV):
            dst = lax.rem(my_i + step, N_DEV)
            rdma = pltpu.make_async_remote_copy(
                src_ref=x_ref.at[pl.ds(dst * BLK, BLK), :],
                dst_ref=gather_ref.at[my_i],
                send_sem=send_sems.at[step],
                recv_sem=recv_sems.at[step],
                device_id=(dst,),
                device_id_type=pl.DeviceIdType.MESH,
            )
            rdma.start()
            sends.append(rdma)

        gather_ref[my_i] = x_ref[pl.ds(my_i * BLK, BLK), :]

        for step in range(1, N_DEV):
            src = lax.rem(my_i + N_DEV - step, N_DEV)
            recv = pltpu.make_async_remote_copy(
                src_ref=x_ref.at[pl.ds(0, BLK), :],
                dst_ref=gather_ref.at[src],
                send_sem=send_sems.at[step],
                recv_sem=recv_sems.at[step],
                device_id=(my_i,),
                device_id_type=pl.DeviceIdType.MESH,
            )
            recv.wait_recv()
        for rdma in sends:
            rdma.wait_send()
        w_copy.wait()

        g = gather_ref[...]
        xrow = jnp.transpose(g, (1, 0, 2)).reshape(BLK, k)
        y = jnp.dot(xrow, w_vmem_ref[...], preferred_element_type=jnp.float32)
        out_ref[...] = y * jax.nn.sigmoid(y)

    return pl.pallas_call(
        body,
        out_shape=jax.ShapeDtypeStruct((BLK, n), jnp.float32),
        in_specs=[
            pl.BlockSpec(memory_space=pltpu.VMEM),
            pl.BlockSpec(memory_space=pl.ANY),
        ],
        out_specs=pl.BlockSpec(memory_space=pltpu.VMEM),
        scratch_shapes=[
            pltpu.VMEM((N_DEV, BLK, BLK), jnp.float32),
            pltpu.VMEM((1024, 1024), jnp.float32),
            pltpu.SemaphoreType.DMA((N_DEV,)),
            pltpu.SemaphoreType.DMA((N_DEV,)),
            pltpu.SemaphoreType.DMA,
        ],
        compiler_params=pltpu.CompilerParams(collective_id=0),
    )(x, w_mat)
